# baseline (speedup 1.0000x reference)
import functools

import jax
import jax.numpy as jnp
from jax import lax
from jax.experimental import pallas as pl
from jax.experimental.pallas import tpu as pltpu
from jax.experimental.pallas import tpu_sc as plsc

TOTAL_TOK = 32768
D_TS = 16
D_FEAT = 112
IN_FEATS = D_TS + D_FEAT
HIDDEN = 64
OUT = 32
NUM_SEGMENTS = 16
RRELU_SLOPE = (1.0 / 8.0 + 1.0 / 3.0) / 2.0

BLK = 4096
NB = TOTAL_TOK // BLK

LANES = 16

PART_COLS = 48


def _rrelu(x):
    return jnp.maximum(x, RRELU_SLOPE * x)


def _mlp_partials_body(seg_ref, ts_hbm, ft_hbm, w1t_ref, b1_ref, w2t_ref, b2_ref,
                       part_ref, ts_buf, ft_buf, ts_sem, ft_sem):
    i = pl.program_id(0)
    slot = lax.rem(i, 2)
    nxt = lax.rem(i + 1, 2)

    def _start(j, buf_slot):
        pltpu.make_async_copy(
            ts_hbm.at[:, pl.ds(j * BLK, BLK)], ts_buf.at[buf_slot], ts_sem.at[buf_slot]
        ).start()
        pltpu.make_async_copy(
            ft_hbm.at[:, pl.ds(j * BLK, BLK)], ft_buf.at[buf_slot], ft_sem.at[buf_slot]
        ).start()

    @pl.when(i == 0)
    def _():
        _start(i, slot)

    @pl.when(i + 1 < NB)
    def _():
        _start(i + 1, nxt)

    pltpu.make_async_copy(
        ts_hbm.at[:, pl.ds(i * BLK, BLK)], ts_buf.at[slot], ts_sem.at[slot]
    ).wait()
    pltpu.make_async_copy(
        ft_hbm.at[:, pl.ds(i * BLK, BLK)], ft_buf.at[slot], ft_sem.at[slot]
    ).wait()

    h = jnp.dot(w1t_ref[:, 0:D_TS], ts_buf[slot], preferred_element_type=jnp.float32)
    h = h + jnp.dot(w1t_ref[:, D_TS:], ft_buf[slot], preferred_element_type=jnp.float32)
    b1_col = b1_ref[...].reshape(HIDDEN, 1)
    b2_col = b2_ref[...].reshape(OUT, 1)
    h = _rrelu(h + b1_col)
    y = _rrelu(jnp.dot(w2t_ref[...], h, preferred_element_type=jnp.float32)
               + b2_col)
    ids = seg_ref[0]
    seg_iota = lax.broadcasted_iota(jnp.int32, (NUM_SEGMENTS, BLK), 0)
    onehot_t = (seg_iota == ids).astype(jnp.float32)
    part = lax.dot_general(
        onehot_t, y, (((1,), (1,)), ((), ())), preferred_element_type=jnp.float32
    )
    cnt_col = jnp.sum(onehot_t, axis=1).reshape(NUM_SEGMENTS, 1)
    pad = jnp.zeros((NUM_SEGMENTS, PART_COLS - OUT - 1), jnp.float32)
    part_ref[0] = jnp.concatenate([part, cnt_col, pad], axis=1)


def _mlp_partials(seg3, ts_t, feats_t, W1t, b1_row, W2t, b2_row):
    return pl.pallas_call(
        _mlp_partials_body,
        grid=(NB,),
        in_specs=[
            pl.BlockSpec((1, 1, BLK), lambda i: (i, 0, 0)),
            pl.BlockSpec(memory_space=pltpu.HBM),
            pl.BlockSpec(memory_space=pltpu.HBM),
            pl.BlockSpec((HIDDEN, IN_FEATS), lambda i: (0, 0)),
            pl.BlockSpec((1, HIDDEN), lambda i: (0, 0)),
            pl.BlockSpec((OUT, HIDDEN), lambda i: (0, 0)),
            pl.BlockSpec((1, OUT), lambda i: (0, 0)),
        ],
        out_specs=pl.BlockSpec((1, NUM_SEGMENTS, PART_COLS), lambda i: (i, 0, 0)),
        out_shape=jax.ShapeDtypeStruct((NB, NUM_SEGMENTS, PART_COLS), jnp.float32),
        scratch_shapes=[
            pltpu.VMEM((2, D_TS, BLK), jnp.float32),
            pltpu.VMEM((2, D_FEAT, BLK), jnp.float32),
            pltpu.SemaphoreType.DMA((2,)),
            pltpu.SemaphoreType.DMA((2,)),
        ],
    )(seg3, ts_t, feats_t, W1t, b1_row, W2t, b2_row)


@functools.partial(
    pl.kernel,
    mesh=plsc.VectorSubcoreMesh(core_axis_name="c", subcore_axis_name="s"),
    out_type=jax.ShapeDtypeStruct((NUM_SEGMENTS, OUT), jnp.float32),
    scratch_types=[
        pltpu.VMEM((NB, PART_COLS), jnp.float32),
        pltpu.VMEM((OUT,), jnp.float32),
    ],
)
def _sc_merge(parts_hbm, out_hbm, row_v, out_row_v):
    cid = lax.axis_index("c")
    sid = lax.axis_index("s")
    wid = sid * 2 + cid

    @pl.when(wid < NUM_SEGMENTS)
    def _():
        pltpu.sync_copy(parts_hbm.at[:, wid, :], row_v)
        cchunk = row_v[0, pl.ds(OUT, LANES)]
        for b in range(1, NB):
            cchunk = cchunk + row_v[b, pl.ds(OUT, LANES)]
        c = jnp.maximum(cchunk[0], 1.0)
        for k in range(OUT // LANES):
            acc = row_v[0, pl.ds(k * LANES, LANES)]
            for b in range(1, NB):
                acc = acc + row_v[b, pl.ds(k * LANES, LANES)]
            out_row_v[pl.ds(k * LANES, LANES)] = acc / c
        pltpu.sync_copy(out_row_v, out_hbm.at[wid, :])


def kernel(ts, feats, segment_ids, W1, b1, W2, b2):
    seg3 = segment_ids.reshape(NB, 1, BLK)
    ts_t = pltpu.with_memory_space_constraint(ts.T, pltpu.HBM)
    feats_t = pltpu.with_memory_space_constraint(feats.T, pltpu.HBM)
    partials = _mlp_partials(
        seg3, ts_t, feats_t, W1.T, b1.reshape(1, HIDDEN), W2.T, b2.reshape(1, OUT)
    )
    return _sc_merge(partials)


# speedup vs baseline: 1.0802x; 1.0802x over previous
import functools

import jax
import jax.numpy as jnp
from jax import lax
from jax.experimental import pallas as pl
from jax.experimental.pallas import tpu as pltpu
from jax.experimental.pallas import tpu_sc as plsc

TOTAL_TOK = 32768
D_TS = 16
D_FEAT = 112
IN_FEATS = D_TS + D_FEAT
HIDDEN = 64
OUT = 32
NUM_SEGMENTS = 16
RRELU_SLOPE = (1.0 / 8.0 + 1.0 / 3.0) / 2.0

BLK = 8192
NB = TOTAL_TOK // BLK

LANES = 16

PART_COLS = 48


def _rrelu(x):
    return jnp.maximum(x, RRELU_SLOPE * x)


def _mlp_partials_body(seg_ref, ts_hbm, ft_hbm, w1t_ref, b1_ref, w2t_ref, b2_ref,
                       part_ref, ts_buf, ft_buf, ts_sem, ft_sem):
    i = pl.program_id(0)
    slot = lax.rem(i, 2)
    nxt = lax.rem(i + 1, 2)

    def _start(j, buf_slot):
        pltpu.make_async_copy(
            ts_hbm.at[:, pl.ds(j * BLK, BLK)], ts_buf.at[buf_slot], ts_sem.at[buf_slot]
        ).start()
        pltpu.make_async_copy(
            ft_hbm.at[:, pl.ds(j * BLK, BLK)], ft_buf.at[buf_slot], ft_sem.at[buf_slot]
        ).start()

    @pl.when(i == 0)
    def _():
        _start(i, slot)

    @pl.when(i + 1 < NB)
    def _():
        _start(i + 1, nxt)

    pltpu.make_async_copy(
        ts_hbm.at[:, pl.ds(i * BLK, BLK)], ts_buf.at[slot], ts_sem.at[slot]
    ).wait()
    pltpu.make_async_copy(
        ft_hbm.at[:, pl.ds(i * BLK, BLK)], ft_buf.at[slot], ft_sem.at[slot]
    ).wait()

    h = jnp.dot(w1t_ref[:, 0:D_TS], ts_buf[slot], preferred_element_type=jnp.float32)
    h = h + jnp.dot(w1t_ref[:, D_TS:], ft_buf[slot], preferred_element_type=jnp.float32)
    b1_col = b1_ref[...].reshape(HIDDEN, 1)
    b2_col = b2_ref[...].reshape(OUT, 1)
    h = _rrelu(h + b1_col)
    y = _rrelu(jnp.dot(w2t_ref[...], h, preferred_element_type=jnp.float32)
               + b2_col)
    ids = seg_ref[0]
    seg_iota = lax.broadcasted_iota(jnp.int32, (NUM_SEGMENTS, BLK), 0)
    onehot_t = (seg_iota == ids).astype(jnp.float32)
    part = lax.dot_general(
        onehot_t, y, (((1,), (1,)), ((), ())), preferred_element_type=jnp.float32
    )
    cnt_col = jnp.sum(onehot_t, axis=1).reshape(NUM_SEGMENTS, 1)
    pad = jnp.zeros((NUM_SEGMENTS, PART_COLS - OUT - 1), jnp.float32)
    part_ref[0] = jnp.concatenate([part, cnt_col, pad], axis=1)


def _mlp_partials(seg3, ts_t, feats_t, W1t, b1_row, W2t, b2_row):
    return pl.pallas_call(
        _mlp_partials_body,
        grid=(NB,),
        in_specs=[
            pl.BlockSpec((1, 1, BLK), lambda i: (i, 0, 0)),
            pl.BlockSpec(memory_space=pltpu.HBM),
            pl.BlockSpec(memory_space=pltpu.HBM),
            pl.BlockSpec((HIDDEN, IN_FEATS), lambda i: (0, 0)),
            pl.BlockSpec((1, HIDDEN), lambda i: (0, 0)),
            pl.BlockSpec((OUT, HIDDEN), lambda i: (0, 0)),
            pl.BlockSpec((1, OUT), lambda i: (0, 0)),
        ],
        out_specs=pl.BlockSpec((1, NUM_SEGMENTS, PART_COLS), lambda i: (i, 0, 0)),
        out_shape=jax.ShapeDtypeStruct((NB, NUM_SEGMENTS, PART_COLS), jnp.float32),
        scratch_shapes=[
            pltpu.VMEM((2, D_TS, BLK), jnp.float32),
            pltpu.VMEM((2, D_FEAT, BLK), jnp.float32),
            pltpu.SemaphoreType.DMA((2,)),
            pltpu.SemaphoreType.DMA((2,)),
        ],
    )(seg3, ts_t, feats_t, W1t, b1_row, W2t, b2_row)


@functools.partial(
    pl.kernel,
    mesh=plsc.VectorSubcoreMesh(core_axis_name="c", subcore_axis_name="s"),
    out_type=jax.ShapeDtypeStruct((NUM_SEGMENTS, OUT), jnp.float32),
    scratch_types=[
        pltpu.VMEM((NB, PART_COLS), jnp.float32),
        pltpu.VMEM((OUT,), jnp.float32),
    ],
)
def _sc_merge(parts_hbm, out_hbm, row_v, out_row_v):
    cid = lax.axis_index("c")
    sid = lax.axis_index("s")
    wid = sid * 2 + cid

    @pl.when(wid < NUM_SEGMENTS)
    def _():
        pltpu.sync_copy(parts_hbm.at[:, wid, :], row_v)
        cchunk = row_v[0, pl.ds(OUT, LANES)]
        for b in range(1, NB):
            cchunk = cchunk + row_v[b, pl.ds(OUT, LANES)]
        c = jnp.maximum(cchunk[0], 1.0)
        for k in range(OUT // LANES):
            acc = row_v[0, pl.ds(k * LANES, LANES)]
            for b in range(1, NB):
                acc = acc + row_v[b, pl.ds(k * LANES, LANES)]
            out_row_v[pl.ds(k * LANES, LANES)] = acc / c
        pltpu.sync_copy(out_row_v, out_hbm.at[wid, :])


def kernel(ts, feats, segment_ids, W1, b1, W2, b2):
    seg3 = segment_ids.reshape(NB, 1, BLK)
    ts_t = pltpu.with_memory_space_constraint(ts.T, pltpu.HBM)
    feats_t = pltpu.with_memory_space_constraint(feats.T, pltpu.HBM)
    partials = _mlp_partials(
        seg3, ts_t, feats_t, W1.T, b1.reshape(1, HIDDEN), W2.T, b2.reshape(1, OUT)
    )
    return _sc_merge(partials)


# speedup vs baseline: 1.1899x; 1.1016x over previous
import functools

import jax
import jax.numpy as jnp
from jax import lax
from jax.experimental import pallas as pl
from jax.experimental.pallas import tpu as pltpu
from jax.experimental.pallas import tpu_sc as plsc

TOTAL_TOK = 32768
D_TS = 16
D_FEAT = 112
IN_FEATS = D_TS + D_FEAT
HIDDEN = 64
OUT = 32
NUM_SEGMENTS = 16
RRELU_SLOPE = (1.0 / 8.0 + 1.0 / 3.0) / 2.0

BLK = 16384
NB = TOTAL_TOK // BLK

LANES = 16

PART_COLS = 48


def _rrelu(x):
    return jnp.maximum(x, RRELU_SLOPE * x)


def _mlp_partials_body(seg_ref, ts_ref, ft_ref, w1t_ref, b1_ref, w2t_ref, b2_ref,
                       part_ref):
    h = jnp.dot(w1t_ref[:, 0:D_TS], ts_ref[...], preferred_element_type=jnp.float32)
    h = h + jnp.dot(w1t_ref[:, D_TS:], ft_ref[...], preferred_element_type=jnp.float32)
    b1_col = b1_ref[...].reshape(HIDDEN, 1)
    b2_col = b2_ref[...].reshape(OUT, 1)
    h = _rrelu(h + b1_col)
    y = _rrelu(jnp.dot(w2t_ref[...], h, preferred_element_type=jnp.float32)
               + b2_col)
    ids = seg_ref[0]
    seg_iota = lax.broadcasted_iota(jnp.int32, (NUM_SEGMENTS, BLK), 0)
    onehot_t = (seg_iota == ids).astype(jnp.float32)
    part = lax.dot_general(
        onehot_t, y, (((1,), (1,)), ((), ())), preferred_element_type=jnp.float32
    )
    cnt_col = jnp.sum(onehot_t, axis=1).reshape(NUM_SEGMENTS, 1)
    pad = jnp.zeros((NUM_SEGMENTS, PART_COLS - OUT - 1), jnp.float32)
    part_ref[0] = jnp.concatenate([part, cnt_col, pad], axis=1)


def _mlp_partials(seg3, ts_t, feats_t, W1t, b1_row, W2t, b2_row):
    return pl.pallas_call(
        _mlp_partials_body,
        grid=(NB,),
        in_specs=[
            pl.BlockSpec((1, 1, BLK), lambda i: (i, 0, 0)),
            pl.BlockSpec((D_TS, BLK), lambda i: (0, i)),
            pl.BlockSpec((D_FEAT, BLK), lambda i: (0, i)),
            pl.BlockSpec((HIDDEN, IN_FEATS), lambda i: (0, 0)),
            pl.BlockSpec((1, HIDDEN), lambda i: (0, 0)),
            pl.BlockSpec((OUT, HIDDEN), lambda i: (0, 0)),
            pl.BlockSpec((1, OUT), lambda i: (0, 0)),
        ],
        out_specs=pl.BlockSpec((1, NUM_SEGMENTS, PART_COLS), lambda i: (i, 0, 0)),
        out_shape=jax.ShapeDtypeStruct((NB, NUM_SEGMENTS, PART_COLS), jnp.float32),
    )(seg3, ts_t, feats_t, W1t, b1_row, W2t, b2_row)


@functools.partial(
    pl.kernel,
    mesh=plsc.VectorSubcoreMesh(core_axis_name="c", subcore_axis_name="s", num_cores=1),
    out_type=jax.ShapeDtypeStruct((NUM_SEGMENTS, OUT), jnp.float32),
    scratch_types=[
        pltpu.VMEM((NB, PART_COLS), jnp.float32),
        pltpu.VMEM((OUT,), jnp.float32),
    ],
)
def _sc_merge(parts_hbm, out_hbm, row_v, out_row_v):
    cid = lax.axis_index("c")
    sid = lax.axis_index("s")
    wid = sid + cid * 0

    @pl.when(wid < NUM_SEGMENTS)
    def _():
        pltpu.sync_copy(parts_hbm.at[:, wid, :], row_v)
        cchunk = row_v[0, pl.ds(OUT, LANES)]
        for b in range(1, NB):
            cchunk = cchunk + row_v[b, pl.ds(OUT, LANES)]
        c = jnp.maximum(cchunk[0], 1.0)
        for k in range(OUT // LANES):
            acc = row_v[0, pl.ds(k * LANES, LANES)]
            for b in range(1, NB):
                acc = acc + row_v[b, pl.ds(k * LANES, LANES)]
            out_row_v[pl.ds(k * LANES, LANES)] = acc / c
        pltpu.sync_copy(out_row_v, out_hbm.at[wid, :])


def kernel(ts, feats, segment_ids, W1, b1, W2, b2):
    seg3 = segment_ids.reshape(NB, 1, BLK)
    partials = _mlp_partials(
        seg3, ts.T, feats.T, W1.T, b1.reshape(1, HIDDEN), W2.T, b2.reshape(1, OUT)
    )
    return _sc_merge(partials)


# speedup vs baseline: 1.1938x; 1.0033x over previous
import functools

import jax
import jax.numpy as jnp
from jax import lax
from jax.experimental import pallas as pl
from jax.experimental.pallas import tpu as pltpu
from jax.experimental.pallas import tpu_sc as plsc

TOTAL_TOK = 32768
D_TS = 16
D_FEAT = 112
IN_FEATS = D_TS + D_FEAT
HIDDEN = 64
OUT = 32
NUM_SEGMENTS = 16
RRELU_SLOPE = (1.0 / 8.0 + 1.0 / 3.0) / 2.0

BLK = 16384
NB = TOTAL_TOK // BLK

LANES = 16

PART_COLS = 48


def _rrelu(x):
    return jnp.maximum(x, RRELU_SLOPE * x)


def _mlp_partials_body(seg_ref, ts_ref, ft_ref, w1t_ref, b1_ref, w2t_ref, b2_ref,
                       part_ref):
    h = jnp.dot(w1t_ref[:, 0:D_TS], ts_ref[...], preferred_element_type=jnp.float32)
    h = h + jnp.dot(w1t_ref[:, D_TS:], ft_ref[...], preferred_element_type=jnp.float32)
    b1_col = b1_ref[...].reshape(HIDDEN, 1)
    b2_col = b2_ref[...].reshape(OUT, 1)
    h = _rrelu(h + b1_col)
    y = _rrelu(jnp.dot(w2t_ref[...], h, preferred_element_type=jnp.float32)
               + b2_col)
    ids = seg_ref[0]
    seg_iota = lax.broadcasted_iota(jnp.int32, (NUM_SEGMENTS, BLK), 0)
    onehot_t = (seg_iota == ids).astype(jnp.float32)
    part = lax.dot_general(
        onehot_t, y, (((1,), (1,)), ((), ())), preferred_element_type=jnp.float32
    )
    cnt_col = jnp.sum(onehot_t, axis=1).reshape(NUM_SEGMENTS, 1)
    pad = jnp.zeros((NUM_SEGMENTS, PART_COLS - OUT - 1), jnp.float32)
    part_ref[0] = jnp.concatenate([part, cnt_col, pad], axis=1)


def _mlp_partials(seg3, ts_t, feats_t, W1t, b1_row, W2t, b2_row):
    return pl.pallas_call(
        _mlp_partials_body,
        grid=(NB,),
        in_specs=[
            pl.BlockSpec((1, 1, BLK), lambda i: (i, 0, 0)),
            pl.BlockSpec((D_TS, BLK), lambda i: (0, i)),
            pl.BlockSpec((D_FEAT, BLK), lambda i: (0, i)),
            pl.BlockSpec((HIDDEN, IN_FEATS), lambda i: (0, 0)),
            pl.BlockSpec((1, HIDDEN), lambda i: (0, 0)),
            pl.BlockSpec((OUT, HIDDEN), lambda i: (0, 0)),
            pl.BlockSpec((1, OUT), lambda i: (0, 0)),
        ],
        out_specs=pl.BlockSpec((1, NUM_SEGMENTS, PART_COLS), lambda i: (i, 0, 0)),
        out_shape=jax.ShapeDtypeStruct((NB, NUM_SEGMENTS, PART_COLS), jnp.float32),
    )(seg3, ts_t, feats_t, W1t, b1_row, W2t, b2_row)


@functools.partial(
    pl.kernel,
    mesh=plsc.VectorSubcoreMesh(core_axis_name="c", subcore_axis_name="s", num_cores=1),
    out_type=jax.ShapeDtypeStruct((NUM_SEGMENTS, OUT), jnp.float32),
    scratch_types=[
        pltpu.VMEM((NB, PART_COLS), jnp.float32),
        pltpu.VMEM((OUT,), jnp.float32),
    ],
)
def _sc_merge(parts_hbm, out_hbm, row_v, out_row_v):
    wid = lax.axis_index("s")

    @pl.when(wid < NUM_SEGMENTS)
    def _():
        pltpu.sync_copy(parts_hbm.at[:, wid, :], row_v)
        cchunk = row_v[0, pl.ds(OUT, LANES)]
        for b in range(1, NB):
            cchunk = cchunk + row_v[b, pl.ds(OUT, LANES)]
        c = jnp.maximum(cchunk[0], 1.0)
        for k in range(OUT // LANES):
            acc = row_v[0, pl.ds(k * LANES, LANES)]
            for b in range(1, NB):
                acc = acc + row_v[b, pl.ds(k * LANES, LANES)]
            out_row_v[pl.ds(k * LANES, LANES)] = acc / c
        pltpu.sync_copy(out_row_v, out_hbm.at[wid, :])


def kernel(ts, feats, segment_ids, W1, b1, W2, b2):
    seg3 = segment_ids.reshape(NB, 1, BLK)
    partials = _mlp_partials(
        seg3, ts.T, feats.T, W1.T, b1.reshape(1, HIDDEN), W2.T, b2.reshape(1, OUT)
    )
    return _sc_merge(partials)
